# Initial kernel scaffold; baseline (speedup 1.0000x reference)
import functools

import jax
import jax.numpy as jnp
from jax import lax
from jax.experimental import pallas as pl
from jax.experimental.pallas import tpu as pltpu
from jax.experimental.pallas import tpu_sc as plsc

_B, _N, _D = 32, 1024, 64
_C = 1024


def _argmax_body(x_ref, emb_ref, idx_ref):
    x = x_ref[0]
    xn = x / jnp.maximum(
        jnp.sqrt(jnp.sum(x * x, axis=-1, keepdims=True)), 1e-12)
    e = emb_ref[...]
    en = e / jnp.maximum(
        jnp.sqrt(jnp.sum(e * e, axis=-1, keepdims=True)), 1e-12)
    dist = lax.dot_general(
        xn, en, (((1,), (1,)), ((), ())),
        preferred_element_type=jnp.float32)
    idx_ref[0, 0] = jnp.argmax(dist, axis=-1).astype(jnp.int32)


def _tc_argmax(x, emb):
    out = pl.pallas_call(
        _argmax_body,
        grid=(_B,),
        in_specs=[
            pl.BlockSpec((1, _N, _D), lambda i: (i, 0, 0)),
            pl.BlockSpec((_C, _D), lambda i: (0, 0)),
        ],
        out_specs=pl.BlockSpec((1, 1, _N), lambda i: (i, 0, 0)),
        out_shape=jax.ShapeDtypeStruct((_B, 1, _N), jnp.int32),
    )(x, emb)
    return out.reshape(_B, _N)


def _make_sc_gather():
    nw = 32
    total = _B * _N
    b_per_w = total // nw
    mesh = plsc.VectorSubcoreMesh(core_axis_name="c", subcore_axis_name="s")

    @functools.partial(
        pl.kernel,
        mesh=mesh,
        out_type=jax.ShapeDtypeStruct((total, _D), jnp.float32),
        scratch_types=[
            pltpu.VMEM((b_per_w,), jnp.int32),
            pltpu.VMEM((b_per_w, _D), jnp.float32),
            pltpu.SemaphoreType.DMA,
        ],
    )
    def gather(table_hbm, idx_hbm, out_hbm, idx_v, rows_v, sem):
        wid = lax.axis_index("s") * 2 + lax.axis_index("c")
        base = wid * b_per_w
        pltpu.sync_copy(idx_hbm.at[pl.ds(base, b_per_w)], idx_v)
        pltpu.async_copy(table_hbm.at[idx_v], rows_v, sem).wait()
        pltpu.sync_copy(rows_v, out_hbm.at[pl.ds(base, b_per_w)])

    return gather


_sc_gather = _make_sc_gather()


def kernel(x, embed):
    x = x.astype(jnp.float32)
    emb = embed[0]
    embed_ind = _tc_argmax(x, emb)
    idx_flat = embed_ind.reshape(_B * _N)
    quant = _sc_gather(emb, idx_flat)
    return quant.reshape(_B, _N, _D), embed_ind


# speedup vs baseline: 3.4514x; 3.4514x over previous
import functools

import jax
import jax.numpy as jnp
from jax import lax
from jax.experimental import pallas as pl
from jax.experimental.pallas import tpu as pltpu
from jax.experimental.pallas import tpu_sc as plsc

_B, _N, _D = 32, 1024, 64
_C = 1024


def _argmax_body(x_ref, emb_ref, idx_ref):
    x = x_ref[0]
    xn = x / jnp.maximum(
        jnp.sqrt(jnp.sum(x * x, axis=-1, keepdims=True)), 1e-12)
    e = emb_ref[...]
    en = e / jnp.maximum(
        jnp.sqrt(jnp.sum(e * e, axis=-1, keepdims=True)), 1e-12)
    dist = lax.dot_general(
        xn, en, (((1,), (1,)), ((), ())),
        preferred_element_type=jnp.float32)
    idx_ref[0, 0] = jnp.argmax(dist, axis=-1).astype(jnp.int32)


def _tc_argmax(x, emb):
    out = pl.pallas_call(
        _argmax_body,
        grid=(_B,),
        in_specs=[
            pl.BlockSpec((1, _N, _D), lambda i: (i, 0, 0)),
            pl.BlockSpec((_C, _D), lambda i: (0, 0)),
        ],
        out_specs=pl.BlockSpec((1, 1, _N), lambda i: (i, 0, 0)),
        out_shape=jax.ShapeDtypeStruct((_B, 1, _N), jnp.int32),
    )(x, emb)
    return out.reshape(_B, _N)


_NW = 32
_TOT = _B * _N
_BPW = _TOT // _NW
_CH = 128
_NCH = _BPW // _CH
_DP = 128


@functools.cache
def _make_sc_gather():
    mesh = plsc.VectorSubcoreMesh(
        core_axis_name="c", subcore_axis_name="s",
        num_cores=2, num_subcores=16)

    @functools.partial(
        pl.kernel,
        mesh=mesh,
        out_type=jax.ShapeDtypeStruct((_TOT, _DP), jnp.float32),
        scratch_types=[
            pltpu.VMEM((_NCH, _CH), jnp.int32),
            pltpu.VMEM((2, _CH, _DP), jnp.float32),
            pltpu.SemaphoreType.DMA,
        ],
    )
    def gather(table_hbm, idx_hbm, out_hbm, idx_v, rows_v, sem):
        wid = lax.axis_index("s") * 2 + lax.axis_index("c")
        base = wid * _BPW
        pltpu.sync_copy(idx_hbm.at[wid], idx_v)
        copies = [None, None]
        copies[0] = pltpu.async_copy(
            table_hbm.at[idx_v.at[0]], rows_v.at[0], sem)
        for j in range(_NCH):
            if j + 1 < _NCH:
                copies[(j + 1) % 2] = pltpu.async_copy(
                    table_hbm.at[idx_v.at[j + 1]], rows_v.at[(j + 1) % 2],
                    sem)
            copies[j % 2].wait()
            pltpu.sync_copy(
                rows_v.at[j % 2],
                out_hbm.at[pl.ds(base + j * _CH, _CH)])

    return gather


def kernel(x, embed):
    x = x.astype(jnp.float32)
    emb = embed[0]
    embed_ind = _tc_argmax(x, emb)
    idx3 = embed_ind.reshape(_NW, _NCH, _CH)
    table = jnp.pad(emb, ((0, 0), (0, _DP - _D)))
    quant = _make_sc_gather()(table, idx3)
    return quant[:, :_D].reshape(_B, _N, _D), embed_ind


# speedup vs baseline: 4.5498x; 1.3182x over previous
import functools

import jax
import jax.numpy as jnp
from jax import lax
from jax.experimental import pallas as pl
from jax.experimental.pallas import tpu as pltpu
from jax.experimental.pallas import tpu_sc as plsc

_B, _N, _D = 32, 1024, 64
_C = 1024


def _argmax_body(x_ref, emb_ref, idx_ref):
    x = x_ref[0]
    xn = x / jnp.maximum(
        jnp.sqrt(jnp.sum(x * x, axis=-1, keepdims=True)), 1e-12)
    e = emb_ref[...]
    en = e / jnp.maximum(
        jnp.sqrt(jnp.sum(e * e, axis=-1, keepdims=True)), 1e-12)
    dist = lax.dot_general(
        en, xn, (((1,), (1,)), ((), ())),
        preferred_element_type=jnp.float32)
    idx_ref[0, 0] = jnp.argmax(dist, axis=0).astype(jnp.int32)


def _tc_argmax(x, emb):
    out = pl.pallas_call(
        _argmax_body,
        grid=(_B,),
        in_specs=[
            pl.BlockSpec((1, _N, _D), lambda i: (i, 0, 0)),
            pl.BlockSpec((_C, _D), lambda i: (0, 0)),
        ],
        out_specs=pl.BlockSpec((1, 1, _N), lambda i: (i, 0, 0)),
        out_shape=jax.ShapeDtypeStruct((_B, 1, _N), jnp.int32),
    )(x, emb)
    return out.reshape(_B, _N)


_NW = 32
_TOT = _B * _N
_BPW = _TOT // _NW
_CH = 128
_NCH = _BPW // _CH
_DP = 128


@functools.cache
def _make_sc_gather():
    mesh = plsc.VectorSubcoreMesh(
        core_axis_name="c", subcore_axis_name="s",
        num_cores=2, num_subcores=16)

    @functools.partial(
        pl.kernel,
        mesh=mesh,
        out_type=jax.ShapeDtypeStruct((_TOT, _DP), jnp.float32),
        scratch_types=[
            pltpu.VMEM((_NCH, _CH), jnp.int32),
            pltpu.VMEM((2, _CH, _DP), jnp.float32),
            pltpu.SemaphoreType.DMA,
        ],
    )
    def gather(table_hbm, idx_hbm, out_hbm, idx_v, rows_v, sem):
        wid = lax.axis_index("s") * 2 + lax.axis_index("c")
        base = wid * _BPW
        pltpu.sync_copy(idx_hbm.at[wid], idx_v)
        copies = [None, None]
        copies[0] = pltpu.async_copy(
            table_hbm.at[idx_v.at[0]], rows_v.at[0], sem)
        for j in range(_NCH):
            if j + 1 < _NCH:
                copies[(j + 1) % 2] = pltpu.async_copy(
                    table_hbm.at[idx_v.at[j + 1]], rows_v.at[(j + 1) % 2],
                    sem)
            copies[j % 2].wait()
            pltpu.sync_copy(
                rows_v.at[j % 2],
                out_hbm.at[pl.ds(base + j * _CH, _CH)])

    return gather


def kernel(x, embed):
    x = x.astype(jnp.float32)
    emb = embed[0]
    embed_ind = _tc_argmax(x, emb)
    idx3 = embed_ind.reshape(_NW, _NCH, _CH)
    table = jnp.pad(emb, ((0, 0), (0, _DP - _D)))
    quant = _make_sc_gather()(table, idx3)
    return quant[:, :_D].reshape(_B, _N, _D), embed_ind


# speedup vs baseline: 4.9927x; 1.0974x over previous
import functools

import jax
import jax.numpy as jnp
from jax import lax
from jax.experimental import pallas as pl
from jax.experimental.pallas import tpu as pltpu
from jax.experimental.pallas import tpu_sc as plsc

_B, _N, _D = 32, 1024, 64
_C = 1024


def _argmax_body(x_ref, emb_ref, idx_ref):
    xt = x_ref[0]
    n = jnp.maximum(
        jnp.sqrt(jnp.sum(xt * xt, axis=0, keepdims=True)), 1e-12)
    xn_t = xt / n
    e = emb_ref[...]
    en = e / jnp.maximum(
        jnp.sqrt(jnp.sum(e * e, axis=-1, keepdims=True)), 1e-12)
    dist = lax.dot_general(
        en, xn_t, (((1,), (0,)), ((), ())),
        preferred_element_type=jnp.float32)
    idx_ref[0, 0] = jnp.argmax(dist, axis=0).astype(jnp.int32)


def _tc_argmax(xt, emb):
    nb = xt.shape[0]
    out = pl.pallas_call(
        _argmax_body,
        grid=(nb,),
        in_specs=[
            pl.BlockSpec((1, _D, _N), lambda i: (i, 0, 0)),
            pl.BlockSpec((_C, _D), lambda i: (0, 0)),
        ],
        out_specs=pl.BlockSpec((1, 1, _N), lambda i: (i, 0, 0)),
        out_shape=jax.ShapeDtypeStruct((nb, 1, _N), jnp.int32),
    )(xt, emb)
    return out.reshape(nb, _N)


_NW = 32
_TOT = _B * _N
_BPW = _TOT // _NW
_CH = 128
_NCH = _BPW // _CH
_DP = 128


@functools.cache
def _make_sc_gather(total):
    bpw = total // _NW
    nch = bpw // _CH
    mesh = plsc.VectorSubcoreMesh(
        core_axis_name="c", subcore_axis_name="s",
        num_cores=2, num_subcores=16)

    @functools.partial(
        pl.kernel,
        mesh=mesh,
        out_type=jax.ShapeDtypeStruct((total, _DP), jnp.float32),
        scratch_types=[
            pltpu.VMEM((nch, _CH), jnp.int32),
            pltpu.VMEM((2, _CH, _DP), jnp.float32),
            pltpu.SemaphoreType.DMA,
        ],
    )
    def gather(table_hbm, idx_hbm, out_hbm, idx_v, rows_v, sem):
        wid = lax.axis_index("s") * 2 + lax.axis_index("c")
        base = wid * bpw
        pltpu.sync_copy(idx_hbm.at[wid], idx_v)
        copies = [None, None]
        copies[0] = pltpu.async_copy(
            table_hbm.at[idx_v.at[0]], rows_v.at[0], sem)
        for j in range(nch):
            if j + 1 < nch:
                copies[(j + 1) % 2] = pltpu.async_copy(
                    table_hbm.at[idx_v.at[j + 1]], rows_v.at[(j + 1) % 2],
                    sem)
            copies[j % 2].wait()
            pltpu.sync_copy(
                rows_v.at[j % 2],
                out_hbm.at[pl.ds(base + j * _CH, _CH)])

    return gather


_SPLIT = 2


def kernel(x, embed):
    x = x.astype(jnp.float32)
    emb = embed[0]
    table = jnp.pad(emb, ((0, 0), (0, _DP - _D)))
    xt = jnp.swapaxes(x, 1, 2)
    nb = _B // _SPLIT
    tot = nb * _N
    inds, quants = [], []
    for k in range(_SPLIT):
        ei = _tc_argmax(xt[k * nb:(k + 1) * nb], emb)
        idx3 = ei.reshape(_NW, tot // _NW // _CH, _CH)
        q = _make_sc_gather(tot)(table, idx3)
        inds.append(ei)
        quants.append(q[:, :_D].reshape(nb, _N, _D))
    return (jnp.concatenate(quants, axis=0),
            jnp.concatenate(inds, axis=0))


# speedup vs baseline: 5.6452x; 1.1307x over previous
import functools

import jax
import jax.numpy as jnp
from jax import lax
from jax.experimental import pallas as pl
from jax.experimental.pallas import tpu as pltpu
from jax.experimental.pallas import tpu_sc as plsc

_B, _N, _D = 32, 1024, 64
_C = 1024


def _argmax_body(x_ref, emb_ref, idx_ref):
    xt = x_ref[0]
    n = jnp.maximum(
        jnp.sqrt(jnp.sum(xt * xt, axis=0, keepdims=True)), 1e-12)
    xn_t = xt / n
    et = emb_ref[...]
    en_t = et / jnp.maximum(
        jnp.sqrt(jnp.sum(et * et, axis=0, keepdims=True)), 1e-12)
    dist = lax.dot_general(
        en_t, xn_t, (((0,), (0,)), ((), ())),
        preferred_element_type=jnp.float32)
    idx_ref[0, 0] = jnp.argmax(dist, axis=0).astype(jnp.int32)


def _tc_argmax(x, emb_t):
    xt = jnp.swapaxes(x, 1, 2)
    out = pl.pallas_call(
        _argmax_body,
        grid=(_B,),
        in_specs=[
            pl.BlockSpec((1, _D, _N), lambda i: (i, 0, 0)),
            pl.BlockSpec((_D, _C), lambda i: (0, 0)),
        ],
        out_specs=pl.BlockSpec((1, 1, _N), lambda i: (i, 0, 0)),
        out_shape=jax.ShapeDtypeStruct((_B, 1, _N), jnp.int32),
    )(xt, emb_t)
    return out.reshape(_B, _N)


_NW = 32
_TOT = _B * _N
_BPW = _TOT // _NW
_CH = 128
_NCH = _BPW // _CH
_DP = 128


@functools.cache
def _make_sc_gather():
    mesh = plsc.VectorSubcoreMesh(
        core_axis_name="c", subcore_axis_name="s",
        num_cores=2, num_subcores=16)

    @functools.partial(
        pl.kernel,
        mesh=mesh,
        out_type=jax.ShapeDtypeStruct((_TOT, _DP), jnp.float32),
        scratch_types=[
            pltpu.VMEM((_NCH, _CH), jnp.int32),
            pltpu.VMEM((3, _CH, _DP), jnp.float32),
            pltpu.SemaphoreType.DMA,
            pltpu.SemaphoreType.DMA,
        ],
    )
    def gather(table_hbm, idx_hbm, out_hbm, idx_v, rows_v, gsem, wsem):
        wid = lax.axis_index("s") * 2 + lax.axis_index("c")
        base = wid * _BPW
        pltpu.sync_copy(idx_hbm.at[wid], idx_v)
        gets = [None] * 3
        puts = [None] * 3
        gets[0] = pltpu.async_copy(
            table_hbm.at[idx_v.at[0]], rows_v.at[0], gsem)
        for j in range(_NCH):
            b = j % 3
            if j + 1 < _NCH:
                bn = (j + 1) % 3
                if puts[bn] is not None:
                    puts[bn].wait()
                    puts[bn] = None
                gets[bn] = pltpu.async_copy(
                    table_hbm.at[idx_v.at[j + 1]], rows_v.at[bn], gsem)
            gets[b].wait()
            puts[b] = pltpu.async_copy(
                rows_v.at[b], out_hbm.at[pl.ds(base + j * _CH, _CH)], wsem)
        for p in puts:
            if p is not None:
                p.wait()

    return gather


def kernel(x, embed):
    x = x.astype(jnp.float32)
    emb_t = jnp.swapaxes(embed[0], 0, 1)
    embed_ind = _tc_argmax(x, emb_t)
    idx3 = embed_ind.reshape(_NW, _NCH, _CH)
    table = jnp.pad(embed[0], ((0, 0), (0, _DP - _D)))
    quant = _make_sc_gather()(table, idx3)
    return quant[:, :_D].reshape(_B, _N, _D), embed_ind
